# baseline (speedup 1.0000x reference)
import functools

import jax
import jax.numpy as jnp
from jax import lax
from jax.experimental import pallas as pl
from jax.experimental.pallas import tpu as pltpu
from jax.experimental.pallas import tpu_sc as plsc

_NUM_SC = 2
_NUM_SUBCORES = 16


def _pe_build_kernel(col_ref, row_ref, pe_ref):
    W, half = col_ref.shape
    H = row_ref.shape[0]
    col = col_ref[...]
    row = row_ref[...]
    pe_ref[:, :, :half] = jnp.broadcast_to(col[None, :, :], (H, W, half))
    pe_ref[:, :, half:] = jnp.broadcast_to(row[:, None, :], (H, W, half))


def _build_pe(col_table, row_table, H, W, C):
    return pl.pallas_call(
        _pe_build_kernel,
        out_shape=jax.ShapeDtypeStruct((H, W, C), jnp.float32),
    )(col_table, row_table)


def _sc_fanout(pe, B):
    HW, C = pe.shape
    N_RCHUNK = 8
    N_BGROUP = 4
    r_chunk = HW // N_RCHUNK
    b_group = B // N_BGROUP

    mesh = plsc.VectorSubcoreMesh(
        core_axis_name="c", subcore_axis_name="s",
        num_cores=_NUM_SC, num_subcores=_NUM_SUBCORES)

    @functools.partial(
        pl.kernel,
        out_type=jax.ShapeDtypeStruct((B, HW, C), jnp.float32),
        mesh=mesh,
        scratch_types=[
            pltpu.VMEM((r_chunk, C), jnp.float32),
            pltpu.SemaphoreType.DMA,
            pltpu.SemaphoreType.DMA,
        ],
    )
    def fanout(pe_hbm, out_hbm, slice_v, sem, stage_sem):
        wid = lax.axis_index("s") * _NUM_SC + lax.axis_index("c")
        rchunk_id = lax.rem(wid, N_RCHUNK)
        bgroup_id = lax.div(wid, N_RCHUNK)
        base_r = rchunk_id * r_chunk
        base_b = bgroup_id * b_group

        r_a = (r_chunk // 2) - (r_chunk // 2) % 8
        r_b = r_chunk - r_a
        stage_a = pltpu.make_async_copy(
            pe_hbm.at[pl.ds(base_r, r_a)], slice_v.at[pl.ds(0, r_a)],
            stage_sem)
        stage_b = pltpu.make_async_copy(
            pe_hbm.at[pl.ds(base_r + r_a, r_b)], slice_v.at[pl.ds(r_a, r_b)],
            stage_sem)
        stage_a.start()
        stage_b.start()

        copies = []
        stage_a.wait()
        for k in range(b_group):
            cp = pltpu.make_async_copy(
                slice_v.at[pl.ds(0, r_a)],
                out_hbm.at[base_b + k].at[pl.ds(base_r, r_a)], sem)
            cp.start()
            copies.append(cp)
        stage_b.wait()
        for k in range(b_group):
            cp = pltpu.make_async_copy(
                slice_v.at[pl.ds(r_a, r_b)],
                out_hbm.at[base_b + k].at[pl.ds(base_r + r_a, r_b)], sem)
            cp.start()
            copies.append(cp)
        for cp in copies:
            cp.wait()

    return fanout(pe)


def kernel(image_feature, col_table, row_table):
    B, C, H, W = image_feature.shape
    pe = _build_pe(col_table, row_table, H, W, C)
    out = _sc_fanout(pe.reshape(H * W, C), B)
    return out.reshape(B, H, W, C).transpose(0, 3, 1, 2)


# speedup vs baseline: 1.0172x; 1.0172x over previous
import functools

import jax
import jax.numpy as jnp
from jax import lax
from jax.experimental import pallas as pl
from jax.experimental.pallas import tpu as pltpu
from jax.experimental.pallas import tpu_sc as plsc

_NUM_SC = 2
_NUM_SUBCORES = 16


def _pe_build_kernel(col_ref, row_ref, pe_ref):
    W, half = col_ref.shape
    H = row_ref.shape[0]
    col = col_ref[...]
    row = row_ref[...]
    pe_ref[:, :, :half] = jnp.broadcast_to(col[None, :, :], (H, W, half))
    pe_ref[:, :, half:] = jnp.broadcast_to(row[:, None, :], (H, W, half))


def _build_pe(col_table, row_table, H, W, C):
    return pl.pallas_call(
        _pe_build_kernel,
        out_shape=jax.ShapeDtypeStruct((H, W, C), jnp.float32),
    )(col_table, row_table)


def _sc_fanout(pe, B):
    HW, C = pe.shape
    N_RCHUNK = 8
    N_BGROUP = 4
    r_chunk = HW // N_RCHUNK
    b_group = B // N_BGROUP

    mesh = plsc.VectorSubcoreMesh(
        core_axis_name="c", subcore_axis_name="s",
        num_cores=_NUM_SC, num_subcores=_NUM_SUBCORES)

    @functools.partial(
        pl.kernel,
        out_type=jax.ShapeDtypeStruct((B, HW, C), jnp.float32),
        mesh=mesh,
        scratch_types=[
            pltpu.VMEM((r_chunk, C), jnp.float32),
            pltpu.SemaphoreType.DMA,
        ],
    )
    def fanout(pe_hbm, out_hbm, slice_v, sem):
        wid = lax.axis_index("s") * _NUM_SC + lax.axis_index("c")
        rchunk_id = lax.rem(wid, N_RCHUNK)
        bgroup_id = lax.div(wid, N_RCHUNK)
        base_r = rchunk_id * r_chunk
        base_b = bgroup_id * b_group
        pltpu.sync_copy(pe_hbm.at[pl.ds(base_r, r_chunk)], slice_v)
        copies = [
            pltpu.make_async_copy(
                slice_v, out_hbm.at[base_b + k].at[pl.ds(base_r, r_chunk)],
                sem)
            for k in range(b_group)
        ]
        for cp in copies:
            cp.start()
        for cp in copies:
            cp.wait()

    return fanout(pe)


def kernel(image_feature, col_table, row_table):
    B, C, H, W = image_feature.shape
    pe = _build_pe(col_table, row_table, H, W, C)
    out = _sc_fanout(pe.reshape(H * W, C), B)
    return out.reshape(B, H, W, C).transpose(0, 3, 1, 2)
